# baseline (speedup 1.0000x reference)
import functools

import jax
import jax.numpy as jnp
from jax.experimental import pallas as pl
import jax.experimental.pallas.tpu as pltpu

_N = 4096
_D = 64
_ROWS = 256
_NCHUNK = _N // _ROWS
_NBUF = 4


def _body(x_ref, y_ref, b_ref, vkyw_ref, vkyb_ref, wkxw_ref, wkxb_ref,
          vkxw_ref, vkxb_ref, sig_ref, a_hbm, out_ref, u_ref, abuf, sems):

    def copy(i):
        return pltpu.make_async_copy(
            a_hbm.at[pl.ds(i * _ROWS, _ROWS), :],
            abuf.at[i % _NBUF],
            sems.at[i % _NBUF],
        )

    for j in range(_NBUF):
        copy(j).start()

    cw = vkxw_ref[...] - 2.0 * wkxw_ref[...]
    cb = vkxb_ref[...] - 2.0 * wkxb_ref[...]
    u_ref[...] = (
        jnp.dot(x_ref[...], cw.T, preferred_element_type=jnp.float32) + cb
    )
    u = u_ref[...]
    sig = sig_ref[0, 0]

    for i in range(_NCHUNK):
        copy(i).wait()
        t = b_ref[pl.ds(i * _ROWS, _ROWS), :] + jnp.dot(
            abuf[i % _NBUF], u, preferred_element_type=jnp.float32
        )
        vky = (
            jnp.dot(y_ref[pl.ds(i * _ROWS, _ROWS), :], vkyw_ref[...].T,
                    preferred_element_type=jnp.float32)
            + vkyb_ref[...]
        )
        out_ref[pl.ds(i * _ROWS, _ROWS), :] = jnp.maximum(vky - sig * t, 0.0)
        if i + _NBUF < _NCHUNK:
            copy(i + _NBUF).start()


@functools.partial(jax.jit, static_argnames=())
def kernel(x, y, A, b, Vky_W, Vky_b, Wkx_W, Wkx_b, Vkx_W, Vkx_b, sigma):
    n, d = x.shape

    vmem = lambda: pl.BlockSpec(memory_space=pltpu.VMEM)

    out = pl.pallas_call(
        _body,
        in_specs=[
            vmem(),
            vmem(),
            vmem(),
            vmem(),
            vmem(),
            vmem(),
            vmem(),
            vmem(),
            vmem(),
            pl.BlockSpec(memory_space=pltpu.SMEM),
            pl.BlockSpec(memory_space=pl.ANY),
        ],
        out_specs=vmem(),
        out_shape=jax.ShapeDtypeStruct((n, d), jnp.float32),
        scratch_shapes=[
            pltpu.VMEM((n, d), jnp.float32),
            pltpu.VMEM((_NBUF, _ROWS, n), jnp.float32),
            pltpu.SemaphoreType.DMA((_NBUF,)),
        ],
    )(
        x, y, b,
        Vky_W, Vky_b.reshape(1, d),
        Wkx_W, Wkx_b.reshape(1, d),
        Vkx_W, Vkx_b.reshape(1, d),
        sigma.reshape(1, 1),
        A,
    )
    return out


# speedup vs baseline: 1.0080x; 1.0080x over previous
import functools

import jax
import jax.numpy as jnp
from jax.experimental import pallas as pl
import jax.experimental.pallas.tpu as pltpu


def _body(x_ref, y_ref, a_ref, b_ref, vkyw_ref, vkyb_ref, wkxw_ref,
          wkxb_ref, vkxw_ref, vkxb_ref, sig_ref, out_ref):
    cw = vkxw_ref[...] - 2.0 * wkxw_ref[...]
    cb = vkxb_ref[...] - 2.0 * wkxb_ref[...]
    u = jnp.dot(x_ref[...], cw.T, preferred_element_type=jnp.float32) + cb
    t = b_ref[...] + jnp.dot(
        a_ref[...], u, preferred_element_type=jnp.float32
    )
    vky = (
        jnp.dot(y_ref[...], vkyw_ref[...].T, preferred_element_type=jnp.float32)
        + vkyb_ref[...]
    )
    out_ref[...] = jnp.maximum(vky - sig_ref[0, 0] * t, 0.0)


@functools.partial(jax.jit, static_argnames=())
def kernel(x, y, A, b, Vky_W, Vky_b, Wkx_W, Wkx_b, Vkx_W, Vkx_b, sigma):
    n, d = x.shape
    bm = 512
    grid = (n // bm,)

    full = lambda shape: pl.BlockSpec(shape, lambda i: (0, 0))
    row_blk = lambda w: pl.BlockSpec((bm, w), lambda i: (i, 0))

    out = pl.pallas_call(
        _body,
        grid=grid,
        in_specs=[
            full((n, d)),
            row_blk(d),
            row_blk(n),
            row_blk(1),
            full((d, d)),
            full((1, d)),
            full((d, d)),
            full((1, d)),
            full((d, d)),
            full((1, d)),
            pl.BlockSpec(memory_space=pltpu.SMEM),
        ],
        out_specs=row_blk(d),
        out_shape=jax.ShapeDtypeStruct((n, d), jnp.float32),
        compiler_params=pltpu.CompilerParams(
            dimension_semantics=("parallel",),
        ),
    )(
        x, y, A, b,
        Vky_W, Vky_b.reshape(1, d),
        Wkx_W, Wkx_b.reshape(1, d),
        Vkx_W, Vkx_b.reshape(1, d),
        sigma.reshape(1, 1),
    )
    return out
